# baseline (speedup 1.0000x reference)
import functools

import jax
import jax.numpy as jnp
from jax import lax
from jax.experimental import pallas as pl
from jax.experimental.pallas import tpu as pltpu
from jax.experimental.pallas import tpu_sc as plsc

NUM_FIELDS = 26
FEAT_NUM = 100000
FEATURE_LENGTH = NUM_FIELDS * FEAT_NUM
K = 16
BATCH = 16384

NC = 2
NS = 16
NW = NC * NS

TB = 128
NBLK = FEATURE_LENGTH // TB
TAIL = FEATURE_LENGTH - NBLK * TB
BPT = 635
NJ_LAST = NBLK - (NW - 1) * BPT

BPW = BATCH // NW
CHUNK_B = 64
NCHUNK = BPW // CHUNK_B
CHUNK_ROWS = CHUNK_B * NUM_FIELDS


def _tr_body(vt_hbm, vr_hbm, si_a, si_b, so_a, so_b,
             sem_ia, sem_ib, sem_oa, sem_ob):
    w = lax.axis_index("s") * NC + lax.axis_index("c")
    iota16 = lax.iota(jnp.int32, 16)
    nj = jnp.where(w == NW - 1, NJ_LAST, BPT)
    npair = jnp.where(w == NW - 1, (NJ_LAST + 1) // 2, (BPT + 1) // 2)
    blk0 = w * BPT

    def start_in(j, si, sem):
        pltpu.async_copy(
            vt_hbm.at[:, pl.ds((blk0 + j) * TB, TB)], si, sem)

    def wait_in(j, si, sem):
        pltpu.make_async_copy(
            vt_hbm.at[:, pl.ds((blk0 + j) * TB, TB)], si, sem).wait()

    def start_out(j, so, sem):
        pltpu.async_copy(so, vr_hbm.at[pl.ds((blk0 + j) * TB, TB)], sem)

    def drain_out(so, sem):
        pltpu.make_async_copy(so, vr_hbm.at[pl.ds(0, TB)], sem).wait()

    def transpose(si, so):
        def rbody(r, carry):
            for u in range(8):
                rr = r * 8 + u
                col = jnp.broadcast_to(rr, (16,)).astype(jnp.int32)
                so[rr, :] = plsc.load_gather(si, [iota16, col])
            return carry
        lax.fori_loop(0, TB // 8, rbody, 0)

    start_in(0, si_a, sem_ia)

    def pair(i, carry):
        ja = 2 * i
        jb = 2 * i + 1

        @pl.when(jb < nj)
        def _():
            start_in(jb, si_b, sem_ib)

        wait_in(ja, si_a, sem_ia)

        @pl.when(i > 0)
        def _():
            drain_out(so_a, sem_oa)

        transpose(si_a, so_a)
        start_out(ja, so_a, sem_oa)

        @pl.when(ja + 2 < nj)
        def _():
            start_in(ja + 2, si_a, sem_ia)

        @pl.when(jb < nj)
        def _():
            wait_in(jb, si_b, sem_ib)

            @pl.when(i > 0)
            def _():
                drain_out(so_b, sem_ob)

            transpose(si_b, so_b)
            start_out(jb, so_b, sem_ob)

        return carry

    lax.fori_loop(0, npair, pair, 0)
    drain_out(so_a, sem_oa)
    drain_out(so_b, sem_ob)

    @pl.when(w == NW - 1)
    def _():
        r0 = NBLK * TB
        pltpu.sync_copy(vt_hbm.at[:, pl.ds(r0, TAIL)],
                        si_a.at[:, pl.ds(0, TAIL)])

        def rbody(r, carry):
            col = jnp.broadcast_to(r, (16,)).astype(jnp.int32)
            so_a[r, :] = plsc.load_gather(si_a, [iota16, col])
            return carry

        lax.fori_loop(0, TAIL, rbody, 0)
        pltpu.sync_copy(so_a.at[pl.ds(0, TAIL)], vr_hbm.at[pl.ds(r0, TAIL)])


def _fm_body(inputs_hbm, w0_hbm, w_hbm, vr_hbm, out_hbm,
             idx_v, rows_a, rows_b, wval_v, rbuf_v, outbuf_v, w0_v,
             sem_a, sem_b, sem_w):
    wid = lax.axis_index("s") * NC + lax.axis_index("c")
    iota16 = lax.iota(jnp.int32, 16)
    bbase = wid * BPW

    pltpu.sync_copy(inputs_hbm.at[:, pl.ds(bbase, BPW)], idx_v)
    pltpu.sync_copy(w0_hbm, w0_v.at[pl.ds(0, 1)])

    for f in range(NUM_FIELDS):
        def obody(l, carry, f=f):
            sl = idx_v[f, pl.ds(l * 16, 16)]
            idx_v[f, pl.ds(l * 16, 16)] = sl + f * FEAT_NUM
            return carry
        lax.fori_loop(0, BPW // 16, obody, 0)

    wcps = []
    for f in range(NUM_FIELDS):
        wcps.append(pltpu.async_copy(
            w_hbm.at[idx_v.at[f]], wval_v.at[f], sem_w))

    bufs = (rows_a, rows_b)
    sems = (sem_a, sem_b)

    def fire(c):
        rows = bufs[c % 2]
        sem = sems[c % 2]
        cps = []
        for f in range(NUM_FIELDS):
            cps.append(pltpu.async_copy(
                vr_hbm.at[idx_v.at[f, pl.ds(c * CHUNK_B, CHUNK_B)]],
                rows.at[pl.ds(f * CHUNK_B, CHUNK_B)], sem))
        return cps

    w0s = w0_v[pl.ds(0, 16)][0]

    def compute(c):
        rows = bufs[c % 2]

        def bbody(b, carry):
            r = rows[b, :]
            s = r
            sq = r * r
            for f in range(1, NUM_FIELDS):
                r = rows[f * CHUNK_B + b, :]
                s = s + r
                sq = sq + r * r
            rbuf_v[b, :] = s * s - sq
            return carry

        lax.fori_loop(0, CHUNK_B, bbody, 0)

        def gbody(g, carry):
            bv = g * 16 + iota16
            acc = plsc.load_gather(rbuf_v, [bv, jnp.zeros((16,), jnp.int32)])
            for k in range(1, K):
                acc = acc + plsc.load_gather(
                    rbuf_v, [bv, jnp.full((16,), k, jnp.int32)])
            colv = c * CHUNK_B + g * 16 + iota16
            fo = plsc.load_gather(wval_v, [jnp.zeros((16,), jnp.int32), colv])
            for f in range(1, NUM_FIELDS):
                fo = fo + plsc.load_gather(
                    wval_v, [jnp.full((16,), f, jnp.int32), colv])
            outbuf_v[pl.ds(c * CHUNK_B + g * 16, 16)] = w0s + fo + 0.5 * acc
            return carry

        lax.fori_loop(0, CHUNK_B // 16, gbody, 0)

    pending = fire(0)
    for c in range(NCHUNK):
        nxt = fire(c + 1) if c + 1 < NCHUNK else []
        for cp in pending:
            cp.wait()
        pending = nxt
        if c == 0:
            for cp in wcps:
                cp.wait()
        compute(c)

    pltpu.sync_copy(outbuf_v, out_hbm.at[pl.ds(bbase, BPW)])


@jax.jit
def _fm(inputs_t, w0, w_flat, vt):
    mesh = plsc.VectorSubcoreMesh(core_axis_name="c", subcore_axis_name="s")
    params = pltpu.CompilerParams(
        needs_layout_passes=False, use_tc_tiling_on_sc=False)

    vr = functools.partial(
        pl.kernel,
        out_type=jax.ShapeDtypeStruct((FEATURE_LENGTH, K), jnp.float32),
        mesh=mesh,
        compiler_params=params,
        scratch_types=[
            pltpu.VMEM((16, TB), jnp.float32),
            pltpu.VMEM((16, TB), jnp.float32),
            pltpu.VMEM((TB, 16), jnp.float32),
            pltpu.VMEM((TB, 16), jnp.float32),
            pltpu.SemaphoreType.DMA,
            pltpu.SemaphoreType.DMA,
            pltpu.SemaphoreType.DMA,
            pltpu.SemaphoreType.DMA,
        ],
    )(_tr_body)(vt)

    out = functools.partial(
        pl.kernel,
        out_type=jax.ShapeDtypeStruct((BATCH,), jnp.float32),
        mesh=mesh,
        compiler_params=params,
        scratch_types=[
            pltpu.VMEM((NUM_FIELDS, BPW), jnp.int32),
            pltpu.VMEM((CHUNK_ROWS, K), jnp.float32),
            pltpu.VMEM((CHUNK_ROWS, K), jnp.float32),
            pltpu.VMEM((NUM_FIELDS, BPW), jnp.float32),
            pltpu.VMEM((CHUNK_B, K), jnp.float32),
            pltpu.VMEM((BPW,), jnp.float32),
            pltpu.VMEM((16,), jnp.float32),
            pltpu.SemaphoreType.DMA,
            pltpu.SemaphoreType.DMA,
            pltpu.SemaphoreType.DMA,
        ],
    )(_fm_body)(inputs_t, w0, w_flat, vr)
    return out


def kernel(inputs, w0, w, V):
    out = _fm(inputs.T, w0, w.reshape(FEATURE_LENGTH), V.T)
    return out.reshape(BATCH, 1)
